# baseline (speedup 1.0000x reference)
import dataclasses
import functools

import jax
import jax.numpy as jnp
from jax import lax
from jax.experimental import pallas as pl
from jax.experimental.pallas import tpu as pltpu
from jax.experimental.pallas import tpu_sc as plsc

NC = 2
NS = 16
NW = NC * NS
CHUNK = 128

def _mesh():
    return plsc.VectorSubcoreMesh(core_axis_name="c", subcore_axis_name="s")


def _sc_params():
    cp = pltpu.CompilerParams()
    if "needs_layout_passes" in pltpu.CompilerParams.__dataclass_fields__:
        cp = dataclasses.replace(cp, needs_layout_passes=False)
    return cp


def _deg_kernel(n_pad, cpt):
    deg_rows = n_pad // 16

    @functools.partial(
        pl.kernel,
        mesh=_mesh(),
        out_type=jax.ShapeDtypeStruct((NW, deg_rows, 16), jnp.float32),
        scratch_types=[
            pltpu.VMEM((cpt, CHUNK), jnp.int32),
            pltpu.VMEM((deg_rows, 16), jnp.float32),
        ],
        compiler_params=_sc_params(),
    )
    def deg_kernel(dst_hbm, zeros_hbm, out_hbm, idx_v, deg_v):
        cid = lax.axis_index("c")
        sid = lax.axis_index("s")
        wid = cid * NS + sid
        pltpu.sync_copy(zeros_hbm, deg_v)
        pltpu.sync_copy(dst_hbm.at[wid], idx_v)
        ones = jnp.ones((16,), jnp.float32)

        @pl.loop(0, cpt)
        def _(c):
            @pl.loop(0, CHUNK // 16)
            def _(j):
                idx = idx_v[c, pl.ds(j * 16, 16)]
                row = lax.shift_right_logical(idx, 4)
                col = lax.bitwise_and(idx, 15)
                plsc.addupdate_scatter(deg_v, [row, col], ones)

        pltpu.sync_copy(deg_v, out_hbm.at[wid])

    return deg_kernel


def _dinv_body(degp_ref, dinv_ref):
    deg = jnp.sum(degp_ref[...], axis=0) + 1.0
    dinv_ref[...] = lax.rsqrt(deg)


def _seg_kernel(n_pad, cpt, rows, d):
    hc = cpt // 2

    @functools.partial(
        pl.kernel,
        mesh=_mesh(),
        out_type=jax.ShapeDtypeStruct((NC, n_pad, d), jnp.float32),
        scratch_types=[
            pltpu.VMEM_SHARED((n_pad, d), jnp.float32),
            pltpu.VMEM((hc, CHUNK), jnp.int32),
            pltpu.VMEM((hc, CHUNK), jnp.int32),
            pltpu.VMEM((CHUNK, d), jnp.float32),
            pltpu.VMEM((CHUNK, d), jnp.float32),
        ],
    )
    def seg_kernel(y_hbm, src_hbm, dst_hbm, zeros_hbm, out_hbm,
                   acc, src_v, dst_v, buf0, buf1):
        cid = lax.axis_index("c")
        sid = lax.axis_index("s")
        wid = cid * NS + sid
        pltpu.sync_copy(zeros_hbm, acc.at[pl.ds(sid * rows, rows)])
        plsc.subcore_barrier()

        def phases(sem0, sem1):
            for h in range(2):
                pltpu.sync_copy(src_hbm.at[wid].at[pl.ds(h * hc, hc)], src_v)
                pltpu.sync_copy(dst_hbm.at[wid].at[pl.ds(h * hc, hc)], dst_v)

                @pl.loop(0, hc, step=2)
                def _(c):
                    cp_a = pltpu.make_async_copy(
                        y_hbm.at[src_v.at[c]], buf0, sem0)
                    cp_b = pltpu.make_async_copy(
                        y_hbm.at[src_v.at[c + 1]], buf1, sem1)
                    cp_a.start()
                    cp_b.start()
                    cp_a.wait()
                    pltpu.sync_copy(buf0, acc.at[dst_v.at[c]], add=True)
                    cp_b.wait()
                    pltpu.sync_copy(buf1, acc.at[dst_v.at[c + 1]], add=True)

        pl.run_scoped(phases,
                      sem0=pltpu.SemaphoreType.DMA,
                      sem1=pltpu.SemaphoreType.DMA)
        plsc.subcore_barrier()
        pltpu.sync_copy(
            acc.at[pl.ds(sid * rows, rows)],
            out_hbm.at[cid].at[pl.ds(sid * rows, rows)],
        )

    return seg_kernel


def _tc1_body(dinv_ref, x_ref, w_ref, y_ref):
    xw = lax.dot(x_ref[...], w_ref[...],
                 precision=lax.Precision.HIGHEST,
                 preferred_element_type=jnp.float32)
    y_ref[...] = xw * dinv_ref[...]


def _tc2_body(accp_ref, y1_ref, dinv_ref, b1_ref, w_ref, y2_ref):
    dinv = dinv_ref[...]
    agg = accp_ref[0] + accp_ref[1] + y1_ref[...]
    h = jnp.maximum(agg * dinv + b1_ref[...], 0.0)
    hw = lax.dot(h, w_ref[...],
                 precision=lax.Precision.HIGHEST,
                 preferred_element_type=jnp.float32)
    y2_ref[...] = hw * dinv


def _tc3_body(accp_ref, y2_ref, dinv_ref, b2_ref, out_ref):
    agg = accp_ref[0] + accp_ref[1] + y2_ref[...]
    out_ref[...] = agg * dinv_ref[...] + b2_ref[...]


def kernel(x, edge, W1, b1, W2, b2):
    n, d = x.shape
    h_dim = W1.shape[1]
    e = edge.shape[1]

    cpt = -(-e // (NW * CHUNK))
    cpt = cpt + (-cpt % 16)
    e_pad = NW * cpt * CHUNK
    n_pad = -(-n // (NS * CHUNK)) * (NS * CHUNK)
    rows = n_pad // NS

    src = edge[0].astype(jnp.int32)
    dst = edge[1].astype(jnp.int32)
    pad = e_pad - e
    src_p = jnp.concatenate([src, jnp.zeros((pad,), jnp.int32)])
    dst_p = jnp.concatenate([dst, jnp.full((pad,), n, jnp.int32)])
    src_p = src_p.reshape(NW, cpt, CHUNK)
    dst_p = dst_p.reshape(NW, cpt, CHUNK)

    deg_rows = n_pad // 16
    zeros16 = jnp.zeros((deg_rows, 16), jnp.float32)
    zeros_d = jnp.zeros((rows, d), jnp.float32)

    deg_fn = _deg_kernel(n_pad, cpt)
    seg_fn = _seg_kernel(n_pad, cpt, rows, d)

    degp = deg_fn(dst_p, zeros16)
    dinv2d = pl.pallas_call(
        _dinv_body,
        in_specs=[pl.BlockSpec((NW, deg_rows, 16), lambda: (0, 0, 0))],
        out_specs=pl.BlockSpec((deg_rows, 16), lambda: (0, 0)),
        out_shape=jax.ShapeDtypeStruct((deg_rows, 16), jnp.float32),
    )(degp)
    dinv = dinv2d.reshape(n_pad, 1)[:n]

    grid = (n // 1000,)
    blk = 1000
    w_spec = pl.BlockSpec((d, h_dim), lambda i: (0, 0))
    accp_spec = pl.BlockSpec((NC, blk, d), lambda i: (0, i, 0))
    row_spec = pl.BlockSpec((blk, d), lambda i: (i, 0))
    dinv_spec = pl.BlockSpec((blk, 1), lambda i: (i, 0))
    bias_spec = pl.BlockSpec((1, d), lambda i: (0, 0))

    y1 = pl.pallas_call(
        _tc1_body,
        grid=grid,
        in_specs=[dinv_spec, row_spec, w_spec],
        out_specs=row_spec,
        out_shape=jax.ShapeDtypeStruct((n, h_dim), jnp.float32),
    )(dinv, x, W1)

    accp1 = seg_fn(y1, src_p, dst_p, zeros_d)

    y2 = pl.pallas_call(
        _tc2_body,
        grid=grid,
        in_specs=[accp_spec, row_spec, dinv_spec, bias_spec, w_spec],
        out_specs=row_spec,
        out_shape=jax.ShapeDtypeStruct((n, h_dim), jnp.float32),
    )(accp1[:, :n, :], y1, dinv, b1.reshape(1, d), W2)

    accp2 = seg_fn(y2, src_p, dst_p, zeros_d)

    out = pl.pallas_call(
        _tc3_body,
        grid=grid,
        in_specs=[accp_spec, row_spec, dinv_spec, bias_spec],
        out_specs=row_spec,
        out_shape=jax.ShapeDtypeStruct((n, h_dim), jnp.float32),
    )(accp2[:, :n, :], y2, dinv, b2.reshape(1, d))

    return out


# speedup vs baseline: 1.8438x; 1.8438x over previous
import dataclasses
import functools

import jax
import jax.numpy as jnp
from jax import lax
from jax.experimental import pallas as pl
from jax.experimental.pallas import tpu as pltpu
from jax.experimental.pallas import tpu_sc as plsc

NC = 2
NS = 16
NW = NC * NS
CHUNK = 128

def _mesh():
    return plsc.VectorSubcoreMesh(core_axis_name="c", subcore_axis_name="s")


def _sc_params():
    cp = pltpu.CompilerParams()
    if "needs_layout_passes" in pltpu.CompilerParams.__dataclass_fields__:
        cp = dataclasses.replace(cp, needs_layout_passes=False)
    return cp


def _deg_kernel(n_pad, cpt0):
    deg_rows = n_pad // 16

    @functools.partial(
        pl.kernel,
        mesh=_mesh(),
        out_type=jax.ShapeDtypeStruct((NW, deg_rows, 16), jnp.float32),
        scratch_types=[
            pltpu.VMEM((cpt0, CHUNK), jnp.int32),
            pltpu.VMEM((deg_rows, 16), jnp.float32),
        ],
        compiler_params=_sc_params(),
    )
    def deg_kernel(dst_hbm, zeros_hbm, out_hbm, idx_v, deg_v):
        cid = lax.axis_index("c")
        sid = lax.axis_index("s")
        wid = cid * NS + sid
        pltpu.sync_copy(zeros_hbm, deg_v)
        pltpu.sync_copy(dst_hbm.at[wid], idx_v)
        ones = jnp.ones((16,), jnp.float32)

        @pl.loop(0, cpt0)
        def _(c):
            @pl.loop(0, CHUNK // 16)
            def _(j):
                idx = idx_v[c, pl.ds(j * 16, 16)]
                row = lax.shift_right_logical(idx, 4)
                col = lax.bitwise_and(idx, 15)
                plsc.addupdate_scatter(deg_v, [row, col], ones)

        pltpu.sync_copy(deg_v, out_hbm.at[wid])

    return deg_kernel


def _dinv_body(degp_ref, dinv_ref):
    deg = jnp.sum(degp_ref[...], axis=0) + 1.0
    dinv_ref[...] = lax.rsqrt(deg)


def _seg_kernel(n_pad, cpt0, cpt1, rows, d):
    @functools.partial(
        pl.kernel,
        mesh=_mesh(),
        out_type=jax.ShapeDtypeStruct((NC, n_pad, d), jnp.float32),
        scratch_types=[
            pltpu.VMEM_SHARED((n_pad, d), jnp.float32),
            pltpu.VMEM((cpt0, CHUNK), jnp.int32),
            pltpu.VMEM((cpt0, CHUNK), jnp.int32),
            pltpu.VMEM((CHUNK, d), jnp.float32),
        ],
    )
    def seg_kernel(y_hbm, src_hbm, dst_hbm, zeros_hbm, out_hbm,
                   acc, src_v, dst_v, buf0):
        cid = lax.axis_index("c")
        sid = lax.axis_index("s")
        wid = cid * NS + sid
        nc = jnp.where(cid == 0, cpt0, cpt1)
        pltpu.sync_copy(zeros_hbm, acc.at[pl.ds(sid * rows, rows)])
        pltpu.sync_copy(src_hbm.at[wid], src_v)
        pltpu.sync_copy(dst_hbm.at[wid], dst_v)
        plsc.subcore_barrier()

        @pl.loop(0, nc)
        def _(c):
            pltpu.sync_copy(y_hbm.at[src_v.at[c]], buf0)
            pltpu.sync_copy(buf0, acc.at[dst_v.at[c]], add=True)

        plsc.subcore_barrier()
        pltpu.sync_copy(
            acc.at[pl.ds(sid * rows, rows)],
            out_hbm.at[cid].at[pl.ds(sid * rows, rows)],
        )

    return seg_kernel


def _tc1_body(dinv_ref, x_ref, w_ref, y_ref):
    xw = lax.dot(x_ref[...], w_ref[...],
                 precision=lax.Precision.HIGHEST,
                 preferred_element_type=jnp.float32)
    y_ref[...] = xw * dinv_ref[...]


def _tc2_body(accp_ref, y1_ref, dinv_ref, b1_ref, w_ref, y2_ref):
    dinv = dinv_ref[...]
    agg = accp_ref[0] + accp_ref[1] + y1_ref[...]
    h = jnp.maximum(agg * dinv + b1_ref[...], 0.0)
    hw = lax.dot(h, w_ref[...],
                 precision=lax.Precision.HIGHEST,
                 preferred_element_type=jnp.float32)
    y2_ref[...] = hw * dinv


def _tc3_body(accp_ref, y2_ref, dinv_ref, b2_ref, out_ref):
    agg = accp_ref[0] + accp_ref[1] + y2_ref[...]
    out_ref[...] = agg * dinv_ref[...] + b2_ref[...]


def kernel(x, edge, W1, b1, W2, b2):
    n, d = x.shape
    h_dim = W1.shape[1]
    e = edge.shape[1]

    n_pad = -(-n // (NS * CHUNK)) * (NS * CHUNK)
    rows = n_pad // NS

    tc_total = -(-e // CHUNK)
    cpt0 = -(-(tc_total * 663 // 1000) // NS)
    cpt1 = max(1, -(-(tc_total - cpt0 * NS) // NS))
    e0 = NS * cpt0 * CHUNK
    e_pad = e0 + NS * cpt1 * CHUNK

    src = edge[0].astype(jnp.int32)
    dst = edge[1].astype(jnp.int32)
    pad = e_pad - e
    src_p = jnp.concatenate([src, jnp.zeros((pad,), jnp.int32)])
    dst_p = jnp.concatenate([dst, jnp.full((pad,), n, jnp.int32)])

    def _percore(flat, fill):
        p0 = flat[:e0].reshape(NS, cpt0, CHUNK)
        p1 = flat[e0:].reshape(NS, cpt1, CHUNK)
        tail = jnp.full((NS, cpt0 - cpt1, CHUNK), fill, jnp.int32)
        return jnp.concatenate([p0, jnp.concatenate([p1, tail], axis=1)],
                               axis=0)

    src_p = _percore(src_p, 0)
    dst_p = _percore(dst_p, n)

    deg_rows = n_pad // 16
    zeros16 = jnp.zeros((deg_rows, 16), jnp.float32)
    zeros_d = jnp.zeros((rows, d), jnp.float32)

    deg_fn = _deg_kernel(n_pad, cpt0)
    seg_fn = _seg_kernel(n_pad, cpt0, cpt1, rows, d)

    degp = deg_fn(dst_p, zeros16)
    dinv2d = pl.pallas_call(
        _dinv_body,
        in_specs=[pl.BlockSpec((NW, deg_rows, 16), lambda: (0, 0, 0))],
        out_specs=pl.BlockSpec((deg_rows, 16), lambda: (0, 0)),
        out_shape=jax.ShapeDtypeStruct((deg_rows, 16), jnp.float32),
    )(degp)
    dinv = dinv2d.reshape(n_pad, 1)[:n]

    grid = (n // 1000,)
    blk = 1000
    w_spec = pl.BlockSpec((d, h_dim), lambda i: (0, 0))
    accp_spec = pl.BlockSpec((NC, blk, d), lambda i: (0, i, 0))
    row_spec = pl.BlockSpec((blk, d), lambda i: (i, 0))
    dinv_spec = pl.BlockSpec((blk, 1), lambda i: (i, 0))
    bias_spec = pl.BlockSpec((1, d), lambda i: (0, 0))

    y1 = pl.pallas_call(
        _tc1_body,
        grid=grid,
        in_specs=[dinv_spec, row_spec, w_spec],
        out_specs=row_spec,
        out_shape=jax.ShapeDtypeStruct((n, h_dim), jnp.float32),
    )(dinv, x, W1)

    accp1 = seg_fn(y1, src_p, dst_p, zeros_d)

    y2 = pl.pallas_call(
        _tc2_body,
        grid=grid,
        in_specs=[accp_spec, row_spec, dinv_spec, bias_spec, w_spec],
        out_specs=row_spec,
        out_shape=jax.ShapeDtypeStruct((n, h_dim), jnp.float32),
    )(accp1[:, :n, :], y1, dinv, b1.reshape(1, d), W2)

    accp2 = seg_fn(y2, src_p, dst_p, zeros_d)

    out = pl.pallas_call(
        _tc3_body,
        grid=grid,
        in_specs=[accp_spec, row_spec, dinv_spec, bias_spec],
        out_specs=row_spec,
        out_shape=jax.ShapeDtypeStruct((n, h_dim), jnp.float32),
    )(accp2[:, :n, :], y2, dinv, b2.reshape(1, d))

    return out


# speedup vs baseline: 1.9508x; 1.0581x over previous
import dataclasses
import functools

import jax
import jax.numpy as jnp
from jax import lax
from jax.experimental import pallas as pl
from jax.experimental.pallas import tpu as pltpu
from jax.experimental.pallas import tpu_sc as plsc

NC = 2
NS = 16
NW = NC * NS
CHUNK = 128

def _mesh():
    return plsc.VectorSubcoreMesh(core_axis_name="c", subcore_axis_name="s")


def _sc_params():
    cp = pltpu.CompilerParams()
    if "needs_layout_passes" in pltpu.CompilerParams.__dataclass_fields__:
        cp = dataclasses.replace(cp, needs_layout_passes=False)
    return cp


def _deg_kernel(n_pad, cpt0):
    deg_rows = n_pad // 16

    @functools.partial(
        pl.kernel,
        mesh=_mesh(),
        out_type=jax.ShapeDtypeStruct((NW, deg_rows, 16), jnp.float32),
        scratch_types=[
            pltpu.VMEM((cpt0, CHUNK), jnp.int32),
            pltpu.VMEM((deg_rows, 16), jnp.float32),
        ],
        compiler_params=_sc_params(),
    )
    def deg_kernel(dst_hbm, zeros_hbm, out_hbm, idx_v, deg_v):
        cid = lax.axis_index("c")
        sid = lax.axis_index("s")
        wid = cid * NS + sid
        pltpu.sync_copy(zeros_hbm, deg_v)
        pltpu.sync_copy(dst_hbm.at[wid], idx_v)
        ones = jnp.ones((16,), jnp.float32)

        @pl.loop(0, cpt0)
        def _(c):
            @pl.loop(0, CHUNK // 16)
            def _(j):
                idx = idx_v[c, pl.ds(j * 16, 16)]
                row = lax.shift_right_logical(idx, 4)
                col = lax.bitwise_and(idx, 15)
                plsc.addupdate_scatter(deg_v, [row, col], ones)

        pltpu.sync_copy(deg_v, out_hbm.at[wid])

    return deg_kernel


def _dinv_body(degp_ref, dinv_ref):
    deg = jnp.sum(degp_ref[...], axis=0) + 1.0
    dinv_ref[...] = lax.rsqrt(deg)


def _seg_kernel(n_pad, cpt0, cpt1, rows, d):
    @functools.partial(
        pl.kernel,
        mesh=_mesh(),
        out_type=jax.ShapeDtypeStruct((NC, n_pad, d), jnp.float32),
        scratch_types=[
            pltpu.VMEM_SHARED((n_pad, d), jnp.float32),
            pltpu.VMEM((cpt0, CHUNK), jnp.int32),
            pltpu.VMEM((cpt0, CHUNK), jnp.int32),
            pltpu.VMEM((CHUNK, d), jnp.float32),
        ],
    )
    def seg_kernel(y_hbm, src_hbm, dst_hbm, zeros_hbm, out_hbm,
                   acc, src_v, dst_v, buf0):
        cid = lax.axis_index("c")
        sid = lax.axis_index("s")
        wid = cid * NS + sid
        nc = jnp.where(cid == 0, cpt0, cpt1)
        pltpu.sync_copy(zeros_hbm, acc.at[pl.ds(sid * rows, rows)])
        pltpu.sync_copy(src_hbm.at[wid], src_v)
        pltpu.sync_copy(dst_hbm.at[wid], dst_v)
        plsc.subcore_barrier()

        @pl.loop(0, nc)
        def _(c):
            pltpu.sync_copy(y_hbm.at[src_v.at[c]], buf0)
            pltpu.sync_copy(buf0, acc.at[dst_v.at[c]], add=True)

        plsc.subcore_barrier()
        pltpu.sync_copy(
            acc.at[pl.ds(sid * rows, rows)],
            out_hbm.at[cid].at[pl.ds(sid * rows, rows)],
        )

    return seg_kernel


def _tc1_body(dinv_ref, x_ref, w_ref, y_ref):
    xw = lax.dot(x_ref[...], w_ref[...],
                 precision=lax.Precision.HIGHEST,
                 preferred_element_type=jnp.float32)
    y_ref[...] = xw * dinv_ref[...]


def _tc2_body(accp_ref, y1_ref, dinv_ref, b1_ref, w_ref, y2_ref):
    dinv = dinv_ref[...]
    agg = accp_ref[0] + accp_ref[1] + y1_ref[...]
    h = jnp.maximum(agg * dinv + b1_ref[...], 0.0)
    hw = lax.dot(h, w_ref[...],
                 precision=lax.Precision.HIGHEST,
                 preferred_element_type=jnp.float32)
    y2_ref[...] = hw * dinv


def _tc3_body(accp_ref, y2_ref, dinv_ref, b2_ref, out_ref):
    agg = accp_ref[0] + accp_ref[1] + y2_ref[...]
    out_ref[...] = agg * dinv_ref[...] + b2_ref[...]


def kernel(x, edge, W1, b1, W2, b2):
    n, d = x.shape
    h_dim = W1.shape[1]
    e = edge.shape[1]

    n_pad = -(-n // (NS * CHUNK)) * (NS * CHUNK)
    rows = n_pad // NS

    tc_total = -(-e // CHUNK)
    cpt0 = -(-(tc_total * 611 // 1000) // NS)
    cpt1 = max(1, -(-(tc_total - cpt0 * NS) // NS))
    e0 = NS * cpt0 * CHUNK
    e_pad = e0 + NS * cpt1 * CHUNK

    src = edge[0].astype(jnp.int32)
    dst = edge[1].astype(jnp.int32)
    pad = e_pad - e
    src_p = jnp.concatenate([src, jnp.zeros((pad,), jnp.int32)])
    dst_p = jnp.concatenate([dst, jnp.full((pad,), n, jnp.int32)])

    def _percore(flat, fill):
        p0 = flat[:e0].reshape(NS, cpt0, CHUNK)
        p1 = flat[e0:].reshape(NS, cpt1, CHUNK)
        tail = jnp.full((NS, cpt0 - cpt1, CHUNK), fill, jnp.int32)
        return jnp.concatenate([p0, jnp.concatenate([p1, tail], axis=1)],
                               axis=0)

    src_p = _percore(src_p, 0)
    dst_p = _percore(dst_p, n)

    deg_rows = n_pad // 16
    zeros16 = jnp.zeros((deg_rows, 16), jnp.float32)
    zeros_d = jnp.zeros((rows, d), jnp.float32)

    deg_fn = _deg_kernel(n_pad, cpt0)
    seg_fn = _seg_kernel(n_pad, cpt0, cpt1, rows, d)

    degp = deg_fn(dst_p, zeros16)
    dinv2d = pl.pallas_call(
        _dinv_body,
        in_specs=[pl.BlockSpec((NW, deg_rows, 16), lambda: (0, 0, 0))],
        out_specs=pl.BlockSpec((deg_rows, 16), lambda: (0, 0)),
        out_shape=jax.ShapeDtypeStruct((deg_rows, 16), jnp.float32),
    )(degp)
    dinv = dinv2d.reshape(n_pad, 1)[:n]

    grid = (n // 1000,)
    blk = 1000
    w_spec = pl.BlockSpec((d, h_dim), lambda i: (0, 0))
    accp_spec = pl.BlockSpec((NC, blk, d), lambda i: (0, i, 0))
    row_spec = pl.BlockSpec((blk, d), lambda i: (i, 0))
    dinv_spec = pl.BlockSpec((blk, 1), lambda i: (i, 0))
    bias_spec = pl.BlockSpec((1, d), lambda i: (0, 0))

    y1 = pl.pallas_call(
        _tc1_body,
        grid=grid,
        in_specs=[dinv_spec, row_spec, w_spec],
        out_specs=row_spec,
        out_shape=jax.ShapeDtypeStruct((n, h_dim), jnp.float32),
    )(dinv, x, W1)

    accp1 = seg_fn(y1, src_p, dst_p, zeros_d)

    y2 = pl.pallas_call(
        _tc2_body,
        grid=grid,
        in_specs=[accp_spec, row_spec, dinv_spec, bias_spec, w_spec],
        out_specs=row_spec,
        out_shape=jax.ShapeDtypeStruct((n, h_dim), jnp.float32),
    )(accp1[:, :n, :], y1, dinv, b1.reshape(1, d), W2)

    accp2 = seg_fn(y2, src_p, dst_p, zeros_d)

    out = pl.pallas_call(
        _tc3_body,
        grid=grid,
        in_specs=[accp_spec, row_spec, dinv_spec, bias_spec],
        out_specs=row_spec,
        out_shape=jax.ShapeDtypeStruct((n, h_dim), jnp.float32),
    )(accp2[:, :n, :], y2, dinv, b2.reshape(1, d))

    return out
